# baseline (speedup 1.0000x reference)
import functools

import jax
import jax.numpy as jnp
from jax import lax
from jax.experimental import pallas as pl
from jax.experimental.pallas import tpu as pltpu
from jax.experimental.pallas import tpu_sc as plsc

B, C, W, H = 4, 96, 224, 224
SC_PLANES = 48
NBANDS = W // 8
BAND = 8 * H
VEC = 16
NCOL = H // VEC
RBLK = 24
NBLK = SC_PLANES // RBLK
TC_RB = 16
SC_BLKS = SC_PLANES // TC_RB
CBLKS = C // TC_RB


def _sc_body(x1, x2, o1, o2, buf0, buf1, acc, sem0, sem1):
    wid = lax.axis_index("s") * 2 + lax.axis_index("c")
    row0 = wid * 8
    bufs = (buf0, buf1)
    sems = (sem0, sem1)

    @pl.when(wid < NBANDS)
    def _():
        def start_block(x, blk, which):
            def sb(r, cr):
                p = blk * RBLK + r
                pb = p // C
                pc = p - pb * C
                pltpu.async_copy(
                    x.at[pb, pc, pl.ds(row0, 8), :],
                    bufs[which].at[r],
                    sems[which])
                return cr
            lax.fori_loop(0, RBLK, sb, 0)

        def wait_block(x, which):
            pltpu.make_async_copy(
                x.at[0, pl.ds(0, RBLK), pl.ds(0, 8), :],
                bufs[which], sems[which]).wait()

        for x, o in ((x1, o1), (x2, o2)):
            @plsc.parallel_loop(0, NCOL, 1, unroll=1)
            def _(g):
                z = jnp.zeros((VEC,), jnp.float32)
                for rr in range(8):
                    acc[rr, pl.ds(g * VEC, VEC)] = z

            start_block(x, 0, 0)
            start_block(x, 1, 1)

            def pair(bb, carry, x=x):
                for ph in range(2):
                    blk = bb * 2 + ph
                    wait_block(x, ph)

                    @plsc.parallel_loop(0, NCOL, 1, unroll=1)
                    def _(g, ph=ph):
                        s = g * VEC
                        for rr in range(8):
                            a = acc[rr, pl.ds(s, VEC)]
                            for r in range(RBLK):
                                v = bufs[ph][r, rr, pl.ds(s, VEC)]
                                a = a + jnp.where(v == 0.0, jnp.float32(1.0),
                                                  jnp.float32(0.0))
                            acc[rr, pl.ds(s, VEC)] = a

                    @pl.when(blk + 2 < NBLK)
                    def _(ph=ph, blk=blk, x=x):
                        start_block(x, blk + 2, ph)
                return carry

            lax.fori_loop(0, NBLK // 2, pair, 0)
            pltpu.sync_copy(acc, o.at[pl.ds(row0, 8), :])


@functools.cache
def _sc_count():
    return pl.kernel(
        _sc_body,
        mesh=plsc.VectorSubcoreMesh(core_axis_name="c", subcore_axis_name="s"),
        out_type=[
            jax.ShapeDtypeStruct((W, H), jnp.float32),
            jax.ShapeDtypeStruct((W, H), jnp.float32),
        ],
        scratch_types=[
            pltpu.VMEM((RBLK, 8, H), jnp.float32),
            pltpu.VMEM((RBLK, 8, H), jnp.float32),
            pltpu.VMEM((8, H), jnp.float32),
            pltpu.SemaphoreType.DMA,
            pltpu.SemaphoreType.DMA,
        ],
    )


def _tc_scan_body(x1_ref, x2_ref, o1_ref, o2_ref):
    @pl.when((pl.program_id(0) == 0) & (pl.program_id(1) == 0))
    def _():
        o1_ref[...] = jnp.zeros_like(o1_ref)
        o2_ref[...] = jnp.zeros_like(o2_ref)

    one = jnp.float32(1.0)
    zero = jnp.float32(0.0)
    o1_ref[...] += jnp.sum(jnp.where(x1_ref[...] == 0.0, one, zero),
                           axis=(0, 1))
    o2_ref[...] += jnp.sum(jnp.where(x2_ref[...] == 0.0, one, zero),
                           axis=(0, 1))


def _tc_scan(x1, x2):
    def imap(g, i):
        j = i + SC_BLKS
        return (j // CBLKS, j % CBLKS, 0, 0)

    return pl.pallas_call(
        _tc_scan_body,
        grid=(1, (B * C - SC_PLANES) // TC_RB),
        in_specs=[
            pl.BlockSpec((1, TC_RB, W, H), imap),
            pl.BlockSpec((1, TC_RB, W, H), imap),
        ],
        out_specs=[
            pl.BlockSpec((W, H), lambda b, i: (0, 0)),
            pl.BlockSpec((W, H), lambda b, i: (0, 0)),
        ],
        out_shape=[
            jax.ShapeDtypeStruct((W, H), jnp.float32),
            jax.ShapeDtypeStruct((W, H), jnp.float32),
        ],
        compiler_params=pltpu.CompilerParams(
            dimension_semantics=("arbitrary", "arbitrary")),
    )(x1, x2)


def _tc_body(a1_ref, b1_ref, a2_ref, b2_ref, out_ref):
    c1 = a1_ref[...] + b1_ref[...]
    c2 = a2_ref[...] + b2_ref[...]
    q1 = c1 / (W * H)
    q2 = c2 / (W * H)
    e1 = -jnp.sum(q1 * jnp.log(q1 + 1e-8), axis=1, keepdims=True)
    e2 = -jnp.sum(q2 * jnp.log(q2 + 1e-8), axis=1, keepdims=True)
    u1 = jnp.where(e1 > 0.0, jnp.float32(1.0), jnp.float32(0.0))
    u2 = jnp.where(e2 > 0.0, jnp.float32(1.0), jnp.float32(0.0))

    def g(s):
        p = s / (256.0 * 256.0)
        return p * jnp.log(p + 1e-8)

    s00 = 256.0 - u1 - u2 + 2.0 * u1 * u2
    h0 = -256.0 * (g(s00) + 255.0 * g(u1))
    hj = -256.0 * (g(u2) + 255.0 * g(jnp.full_like(u2, 256.0)))

    def sl1(d):
        ad = jnp.abs(d)
        return jnp.where(ad < 1.0, 0.5 * d * d, ad - 0.5)

    tot = jnp.sum(sl1((e1 + e2) - h0)) + 255.0 * jnp.sum(sl1(-hj))
    out_ref[0, 0] = tot / (W * 256.0)


def _tc_loss(a1, b1, a2, b2):
    return pl.pallas_call(
        _tc_body,
        out_shape=jax.ShapeDtypeStruct((1, 1), jnp.float32),
        out_specs=pl.BlockSpec(memory_space=pltpu.SMEM),
    )(a1, b1, a2, b2)


def kernel(feature_output, f_5):
    s1, s2 = _sc_count()(feature_output, f_5)
    t1, t2 = _tc_scan(feature_output, f_5)
    out = _tc_loss(s1, t1, s2, t2)
    return out[0, 0]


# speedup vs baseline: 1.0041x; 1.0041x over previous
import functools

import jax
import jax.numpy as jnp
from jax import lax
from jax.experimental import pallas as pl
from jax.experimental.pallas import tpu as pltpu
from jax.experimental.pallas import tpu_sc as plsc

B, C, W, H = 4, 96, 224, 224
SC_PLANES = 64
NBANDS = W // 8
BAND = 8 * H
VEC = 16
NCOL = H // VEC
RBLK = 16
NBLK = SC_PLANES // RBLK
TC_RB = 16
SC_BLKS = SC_PLANES // TC_RB
CBLKS = C // TC_RB


def _sc_body(x1, x2, o1, o2, buf0, buf1, acc, sem0, sem1):
    wid = lax.axis_index("s") * 2 + lax.axis_index("c")
    row0 = wid * 8
    bufs = (buf0, buf1)
    sems = (sem0, sem1)

    @pl.when(wid < NBANDS)
    def _():
        def start_block(x, blk, which):
            def sb(r, cr):
                p = blk * RBLK + r
                pb = p // C
                pc = p - pb * C
                pltpu.async_copy(
                    x.at[pb, pc, pl.ds(row0, 8), :],
                    bufs[which].at[r],
                    sems[which])
                return cr
            lax.fori_loop(0, RBLK, sb, 0)

        def wait_block(x, which):
            pltpu.make_async_copy(
                x.at[0, pl.ds(0, RBLK), pl.ds(0, 8), :],
                bufs[which], sems[which]).wait()

        for x, o in ((x1, o1), (x2, o2)):
            @plsc.parallel_loop(0, NCOL, 1, unroll=1)
            def _(g):
                z = jnp.zeros((VEC,), jnp.float32)
                for rr in range(8):
                    acc[rr, pl.ds(g * VEC, VEC)] = z

            start_block(x, 0, 0)
            start_block(x, 1, 1)

            def pair(bb, carry, x=x):
                for ph in range(2):
                    blk = bb * 2 + ph
                    wait_block(x, ph)

                    @plsc.parallel_loop(0, NCOL, 1, unroll=1)
                    def _(g, ph=ph):
                        s = g * VEC
                        for rr in range(8):
                            a = acc[rr, pl.ds(s, VEC)]
                            for r in range(RBLK):
                                v = bufs[ph][r, rr, pl.ds(s, VEC)]
                                a = a + jnp.where(v == 0.0, jnp.float32(1.0),
                                                  jnp.float32(0.0))
                            acc[rr, pl.ds(s, VEC)] = a

                    @pl.when(blk + 2 < NBLK)
                    def _(ph=ph, blk=blk, x=x):
                        start_block(x, blk + 2, ph)
                return carry

            lax.fori_loop(0, NBLK // 2, pair, 0)
            pltpu.sync_copy(acc, o.at[pl.ds(row0, 8), :])


@functools.cache
def _sc_count():
    return pl.kernel(
        _sc_body,
        mesh=plsc.VectorSubcoreMesh(core_axis_name="c", subcore_axis_name="s"),
        out_type=[
            jax.ShapeDtypeStruct((W, H), jnp.float32),
            jax.ShapeDtypeStruct((W, H), jnp.float32),
        ],
        scratch_types=[
            pltpu.VMEM((RBLK, 8, H), jnp.float32),
            pltpu.VMEM((RBLK, 8, H), jnp.float32),
            pltpu.VMEM((8, H), jnp.float32),
            pltpu.SemaphoreType.DMA,
            pltpu.SemaphoreType.DMA,
        ],
    )


def _tc_scan_body(x1_ref, x2_ref, o1_ref, o2_ref):
    @pl.when((pl.program_id(0) == 0) & (pl.program_id(1) == 0))
    def _():
        o1_ref[...] = jnp.zeros_like(o1_ref)
        o2_ref[...] = jnp.zeros_like(o2_ref)

    one = jnp.float32(1.0)
    zero = jnp.float32(0.0)
    o1_ref[...] += jnp.sum(jnp.where(x1_ref[...] == 0.0, one, zero),
                           axis=(0, 1))
    o2_ref[...] += jnp.sum(jnp.where(x2_ref[...] == 0.0, one, zero),
                           axis=(0, 1))


def _tc_scan(x1, x2):
    def imap(g, i):
        j = i + SC_BLKS
        return (j // CBLKS, j % CBLKS, 0, 0)

    return pl.pallas_call(
        _tc_scan_body,
        grid=(1, (B * C - SC_PLANES) // TC_RB),
        in_specs=[
            pl.BlockSpec((1, TC_RB, W, H), imap),
            pl.BlockSpec((1, TC_RB, W, H), imap),
        ],
        out_specs=[
            pl.BlockSpec((W, H), lambda b, i: (0, 0)),
            pl.BlockSpec((W, H), lambda b, i: (0, 0)),
        ],
        out_shape=[
            jax.ShapeDtypeStruct((W, H), jnp.float32),
            jax.ShapeDtypeStruct((W, H), jnp.float32),
        ],
        compiler_params=pltpu.CompilerParams(
            dimension_semantics=("arbitrary", "arbitrary")),
    )(x1, x2)


def _tc_body(a1_ref, b1_ref, a2_ref, b2_ref, out_ref):
    c1 = a1_ref[...] + b1_ref[...]
    c2 = a2_ref[...] + b2_ref[...]
    q1 = c1 / (W * H)
    q2 = c2 / (W * H)
    e1 = -jnp.sum(q1 * jnp.log(q1 + 1e-8), axis=1, keepdims=True)
    e2 = -jnp.sum(q2 * jnp.log(q2 + 1e-8), axis=1, keepdims=True)
    u1 = jnp.where(e1 > 0.0, jnp.float32(1.0), jnp.float32(0.0))
    u2 = jnp.where(e2 > 0.0, jnp.float32(1.0), jnp.float32(0.0))

    def g(s):
        p = s / (256.0 * 256.0)
        return p * jnp.log(p + 1e-8)

    s00 = 256.0 - u1 - u2 + 2.0 * u1 * u2
    h0 = -256.0 * (g(s00) + 255.0 * g(u1))
    hj = -256.0 * (g(u2) + 255.0 * g(jnp.full_like(u2, 256.0)))

    def sl1(d):
        ad = jnp.abs(d)
        return jnp.where(ad < 1.0, 0.5 * d * d, ad - 0.5)

    tot = jnp.sum(sl1((e1 + e2) - h0)) + 255.0 * jnp.sum(sl1(-hj))
    out_ref[0, 0] = tot / (W * 256.0)


def _tc_loss(a1, b1, a2, b2):
    return pl.pallas_call(
        _tc_body,
        out_shape=jax.ShapeDtypeStruct((1, 1), jnp.float32),
        out_specs=pl.BlockSpec(memory_space=pltpu.SMEM),
    )(a1, b1, a2, b2)


def kernel(feature_output, f_5):
    s1, s2 = _sc_count()(feature_output, f_5)
    t1, t2 = _tc_scan(feature_output, f_5)
    out = _tc_loss(s1, t1, s2, t2)
    return out[0, 0]


# speedup vs baseline: 1.0051x; 1.0010x over previous
import functools

import jax
import jax.numpy as jnp
from jax import lax
from jax.experimental import pallas as pl
from jax.experimental.pallas import tpu as pltpu
from jax.experimental.pallas import tpu_sc as plsc

B, C, W, H = 4, 96, 224, 224
SC_PLANES = 64
NBANDS = W // 8
VEC = 16
NCOL = H // VEC
RBLK = 16
NBLK = SC_PLANES // RBLK
TC_RB = 16
SC_BLKS = SC_PLANES // TC_RB
CBLKS = C // TC_RB


def _sc_body(x1, x2, o1, o2, buf0, buf1, acc, sem0, sem1):
    wid = lax.axis_index("s") * 2 + lax.axis_index("c")
    row0 = wid * 8
    bufs = (buf0, buf1)
    sems = (sem0, sem1)

    @pl.when(wid < NBANDS)
    def _():
        def start_block(x, blk, which):
            def sb(r, cr):
                p = blk * RBLK + r
                pb = p // C
                pc = p - pb * C
                pltpu.async_copy(
                    x.at[pb, pc, pl.ds(row0, 8), :],
                    bufs[which].at[r],
                    sems[which])
                return cr
            lax.fori_loop(0, RBLK, sb, 0)

        def wait_block(x, which):
            pltpu.make_async_copy(
                x.at[0, pl.ds(0, RBLK), pl.ds(0, 8), :],
                bufs[which], sems[which]).wait()

        for x, o in ((x1, o1), (x2, o2)):
            @plsc.parallel_loop(0, NCOL, 1, unroll=1)
            def _(g):
                z = jnp.zeros((VEC,), jnp.float32)
                for rr in range(8):
                    acc[rr, pl.ds(g * VEC, VEC)] = z

            start_block(x, 0, 0)
            start_block(x, 1, 1)

            def pair(bb, carry, x=x):
                for ph in range(2):
                    blk = bb * 2 + ph
                    wait_block(x, ph)

                    @plsc.parallel_loop(0, NCOL, 1, unroll=1)
                    def _(g, ph=ph):
                        s = g * VEC
                        for rr in range(8):
                            a = acc[rr, pl.ds(s, VEC)]
                            for r in range(RBLK):
                                v = bufs[ph][r, rr, pl.ds(s, VEC)]
                                a = a + jnp.where(v == 0.0, jnp.float32(1.0),
                                                  jnp.float32(0.0))
                            acc[rr, pl.ds(s, VEC)] = a

                    @pl.when(blk + 2 < NBLK)
                    def _(ph=ph, blk=blk, x=x):
                        start_block(x, blk + 2, ph)
                return carry

            lax.fori_loop(0, NBLK // 2, pair, 0)
            pltpu.sync_copy(acc, o.at[pl.ds(row0, 8), :])


@functools.cache
def _sc_count():
    return pl.kernel(
        _sc_body,
        mesh=plsc.VectorSubcoreMesh(core_axis_name="c", subcore_axis_name="s"),
        out_type=[
            jax.ShapeDtypeStruct((W, H), jnp.float32),
            jax.ShapeDtypeStruct((W, H), jnp.float32),
        ],
        scratch_types=[
            pltpu.VMEM((RBLK, 8, H), jnp.float32),
            pltpu.VMEM((RBLK, 8, H), jnp.float32),
            pltpu.VMEM((8, H), jnp.float32),
            pltpu.SemaphoreType.DMA,
            pltpu.SemaphoreType.DMA,
        ],
    )


def _tc_scan_body(x1_ref, x2_ref, o1_ref, o2_ref):
    @pl.when((pl.program_id(0) == 0) & (pl.program_id(1) == 0))
    def _():
        o1_ref[...] = jnp.zeros_like(o1_ref)
        o2_ref[...] = jnp.zeros_like(o2_ref)

    one = jnp.float32(1.0)
    zero = jnp.float32(0.0)
    o1_ref[...] += jnp.sum(jnp.where(x1_ref[...] == 0.0, one, zero),
                           axis=(0, 1))
    o2_ref[...] += jnp.sum(jnp.where(x2_ref[...] == 0.0, one, zero),
                           axis=(0, 1))


def _tc_scan(x1, x2):
    def imap(g, i):
        j = i + SC_BLKS
        return (j // CBLKS, j % CBLKS, 0, 0)

    return pl.pallas_call(
        _tc_scan_body,
        grid=(1, (B * C - SC_PLANES) // TC_RB),
        in_specs=[
            pl.BlockSpec((1, TC_RB, W, H), imap),
            pl.BlockSpec((1, TC_RB, W, H), imap),
        ],
        out_specs=[
            pl.BlockSpec((W, H), lambda b, i: (0, 0)),
            pl.BlockSpec((W, H), lambda b, i: (0, 0)),
        ],
        out_shape=[
            jax.ShapeDtypeStruct((W, H), jnp.float32),
            jax.ShapeDtypeStruct((W, H), jnp.float32),
        ],
        compiler_params=pltpu.CompilerParams(
            dimension_semantics=("arbitrary", "arbitrary")),
    )(x1, x2)


def _tc_body(a1_ref, b1_ref, a2_ref, b2_ref, out_ref):
    c1 = a1_ref[...] + b1_ref[...]
    c2 = a2_ref[...] + b2_ref[...]
    q1 = c1 / (W * H)
    q2 = c2 / (W * H)
    e1 = -jnp.sum(q1 * jnp.log(q1 + 1e-8), axis=1, keepdims=True)
    e2 = -jnp.sum(q2 * jnp.log(q2 + 1e-8), axis=1, keepdims=True)
    u1 = jnp.where(e1 > 0.0, jnp.float32(1.0), jnp.float32(0.0))
    u2 = jnp.where(e2 > 0.0, jnp.float32(1.0), jnp.float32(0.0))

    def g(s):
        p = s / (256.0 * 256.0)
        return p * jnp.log(p + 1e-8)

    s00 = 256.0 - u1 - u2 + 2.0 * u1 * u2
    h0 = -256.0 * (g(s00) + 255.0 * g(u1))
    hj = -256.0 * (g(u2) + 255.0 * g(jnp.full_like(u2, 256.0)))

    def sl1(d):
        ad = jnp.abs(d)
        return jnp.where(ad < 1.0, 0.5 * d * d, ad - 0.5)

    tot = jnp.sum(sl1((e1 + e2) - h0)) + 255.0 * jnp.sum(sl1(-hj))
    out_ref[0, 0] = tot / (W * 256.0)


def _tc_loss(a1, b1, a2, b2):
    return pl.pallas_call(
        _tc_body,
        out_shape=jax.ShapeDtypeStruct((1, 1), jnp.float32),
        out_specs=pl.BlockSpec(memory_space=pltpu.SMEM),
    )(a1, b1, a2, b2)


def kernel(feature_output, f_5):
    s1, s2 = _sc_count()(feature_output, f_5)
    t1, t2 = _tc_scan(feature_output, f_5)
    out = _tc_loss(s1, t1, s2, t2)
    return out[0, 0]
